# baseline (speedup 1.0000x reference)
import functools

import jax
import jax.numpy as jnp
from jax import lax
from jax.experimental import pallas as pl
from jax.experimental.pallas import tpu as pltpu
from jax.experimental.pallas import tpu_sc as plsc

HIDDEN = 128
LANES = 16
NJ = HIDDEN // LANES
EPS = 1e-12


def _rsqrt(x):
    i = lax.bitcast_convert_type(x, jnp.int32)
    i = jnp.int32(0x5F3759DF) - (i >> 1)
    y = lax.bitcast_convert_type(i, jnp.float32)
    for _ in range(3):
        y = y * (jnp.float32(1.5) - jnp.float32(0.5) * x * y * y)
    return y


def _tree_sum(vs):
    vs = list(vs)
    while len(vs) > 1:
        nxt = [vs[k] + vs[k + 1] for k in range(0, len(vs) - 1, 2)]
        if len(vs) % 2:
            nxt.append(vs[-1])
        vs = nxt
    return vs[0]


def _make_sc_kernel(n_tokens, seq_len, n_chunks):
    num_cores, num_subcores = 2, 16
    mesh = plsc.VectorSubcoreMesh(
        core_axis_name="c", subcore_axis_name="s",
        num_cores=num_cores, num_subcores=num_subcores)
    nw = num_cores * num_subcores
    per_w = n_tokens // nw

    s1 = min(128, seq_len)
    s2 = seq_len - s1

    def body(ids_hbm, tt_hbm, word_hbm, pos_hbm, type_hbm, gamma_hbm,
             beta_hbm, out_hbm, idx_a, idx_b, tt_v, rows_v, pos_v,
             type_v, gamma_v, beta_v, sem_g, sem_o):
        wid = lax.axis_index("s") * num_cores + lax.axis_index("c")
        base_w = wid * per_w

        pltpu.sync_copy(pos_hbm.at[pl.ds(0, seq_len)], pos_v)
        pltpu.sync_copy(type_hbm, type_v)
        pltpu.sync_copy(gamma_hbm, gamma_v)
        pltpu.sync_copy(beta_hbm, beta_v)

        type0 = [type_v[0, pl.ds(j * LANES, LANES)] for j in range(NJ)]
        type1 = [type_v[1, pl.ds(j * LANES, LANES)] for j in range(NJ)]
        dtype_ = [type1[j] - type0[j] for j in range(NJ)]
        gam = [gamma_v[pl.ds(j * LANES, LANES)] for j in range(NJ)]
        bet = [beta_v[pl.ds(j * LANES, LANES)] for j in range(NJ)]

        @plsc.parallel_loop(0, seq_len, 1, unroll=4)
        def fold_type0(i):
            for j in range(NJ):
                pos_v[i, pl.ds(j * LANES, LANES)] = (
                    pos_v[i, pl.ds(j * LANES, LANES)] + type0[j])

        s2p = 80

        def issue_gather(ci):
            base = base_w + ci * seq_len
            slot = lax.rem(ci, 2)
            buf = lax.rem(ci, 3)
            pltpu.sync_copy(ids_hbm.at[pl.ds(base, s1)],
                            idx_a.at[pl.ds(slot * s1, s1)])
            pltpu.sync_copy(ids_hbm.at[pl.ds(base + s1, s2)],
                            idx_b.at[pl.ds(slot * s2p, s2)])
            pltpu.sync_copy(tt_hbm.at[pl.ds(base, seq_len)],
                            tt_v.at[pl.ds(slot * (seq_len + LANES), seq_len)])
            pltpu.async_copy(word_hbm.at[idx_a.at[pl.ds(slot * s1, s1)]],
                             rows_v.at[buf, pl.ds(0, s1)], sem_g)
            pltpu.async_copy(word_hbm.at[idx_b.at[pl.ds(slot * s2p, s2)]],
                             rows_v.at[buf, pl.ds(s1, s2)], sem_g)

        def wait_gather(ci):
            slot = lax.rem(ci, 2)
            buf = lax.rem(ci, 3)
            pltpu.make_async_copy(
                word_hbm.at[idx_a.at[pl.ds(slot * s1, s1)]],
                rows_v.at[buf, pl.ds(0, s1)], sem_g).wait()
            pltpu.make_async_copy(
                word_hbm.at[idx_b.at[pl.ds(slot * s2p, s2)]],
                rows_v.at[buf, pl.ds(s1, s2)], sem_g).wait()

        def wait_out(ci):
            buf = lax.rem(ci, 3)
            base = base_w + ci * seq_len
            pltpu.make_async_copy(rows_v.at[buf],
                                  out_hbm.at[pl.ds(base, seq_len)],
                                  sem_o).wait()

        issue_gather(0)

        def chunk_body(ci, carry):
            buf = lax.rem(ci, 3)
            slot = lax.rem(ci, 2)
            base = base_w + ci * seq_len

            @pl.when(ci >= 2)
            def _():
                wait_out(ci - 2)

            @pl.when(ci < n_chunks - 1)
            def _():
                issue_gather(ci + 1)

            wait_gather(ci)

            @plsc.parallel_loop(0, seq_len, 1, unroll=4)
            def tok_body(i):
                tf = tt_v[pl.ds(slot * (seq_len + LANES) + i,
                                LANES)][0].astype(jnp.float32)
                e = []
                for j in range(NJ):
                    ej = (rows_v[buf, i, pl.ds(j * LANES, LANES)]
                          + pos_v[i, pl.ds(j * LANES, LANES)]
                          + tf * dtype_[j])
                    e.append(ej)
                s = _tree_sum(e)
                q = _tree_sum([ej * ej for ej in e])
                last = jnp.full((LANES,), LANES - 1, jnp.int32)
                mean = plsc.cumsum(s).at[last].get(
                    mode='promise_in_bounds') * jnp.float32(1.0 / HIDDEN)
                meansq = plsc.cumsum(q).at[last].get(
                    mode='promise_in_bounds') * jnp.float32(1.0 / HIDDEN)
                var = meansq - mean * mean
                rstd = _rsqrt(var + jnp.float32(EPS))
                for j in range(NJ):
                    rows_v[buf, i, pl.ds(j * LANES, LANES)] = (
                        (e[j] - mean) * (rstd * gam[j]) + bet[j])

            pltpu.async_copy(rows_v.at[buf], out_hbm.at[pl.ds(base, seq_len)],
                             sem_o)
            return carry

        lax.fori_loop(0, n_chunks, chunk_body, 0)
        wait_out(n_chunks - 2)
        wait_out(n_chunks - 1)

    return pl.kernel(
        body,
        out_type=jax.ShapeDtypeStruct((n_tokens, HIDDEN), jnp.float32),
        mesh=mesh,
        compiler_params=pltpu.CompilerParams(needs_layout_passes=False),
        scratch_types=[
            pltpu.VMEM((2 * s1,), jnp.int32),
            pltpu.VMEM((2 * 80,), jnp.int32),
            pltpu.VMEM((2 * (seq_len + LANES),), jnp.int32),
            pltpu.VMEM((3, seq_len, HIDDEN), jnp.float32),
            pltpu.VMEM((seq_len, HIDDEN), jnp.float32),
            pltpu.VMEM((2, HIDDEN), jnp.float32),
            pltpu.VMEM((HIDDEN,), jnp.float32),
            pltpu.VMEM((HIDDEN,), jnp.float32),
            pltpu.SemaphoreType.DMA,
            pltpu.SemaphoreType.DMA,
        ],
    )


@jax.jit
def kernel(input_ids, token_type_ids, word_emb, pos_emb, type_emb, gamma,
           beta):
    b, l = input_ids.shape
    n = b * l
    nw = 32
    n_chunks = (n // nw) // l
    ids = input_ids.reshape(n).astype(jnp.int32)
    tt = token_type_ids.reshape(n).astype(jnp.int32)
    k = _make_sc_kernel(n, l, n_chunks)
    out = k(ids, tt, word_emb, pos_emb, type_emb, gamma, beta)
    return out.reshape(b, l, HIDDEN)


# speedup vs baseline: 1.5376x; 1.5376x over previous
import functools

import jax
import jax.numpy as jnp
from jax import lax
from jax.experimental import pallas as pl
from jax.experimental.pallas import tpu as pltpu
from jax.experimental.pallas import tpu_sc as plsc

HIDDEN = 128
LANES = 16
NJ = HIDDEN // LANES
EPS = 1e-12


def _rsqrt(x):
    i = lax.bitcast_convert_type(x, jnp.int32)
    i = jnp.int32(0x5F3759DF) - (i >> 1)
    y = lax.bitcast_convert_type(i, jnp.float32)
    for _ in range(3):
        y = y * (jnp.float32(1.5) - jnp.float32(0.5) * x * y * y)
    return y


def _tree_sum(vs):
    vs = list(vs)
    while len(vs) > 1:
        nxt = [vs[k] + vs[k + 1] for k in range(0, len(vs) - 1, 2)]
        if len(vs) % 2:
            nxt.append(vs[-1])
        vs = nxt
    return vs[0]


def _make_sc_kernel(n_tokens, seq_len, n_chunks):
    num_cores, num_subcores = 2, 16
    mesh = plsc.VectorSubcoreMesh(
        core_axis_name="c", subcore_axis_name="s",
        num_cores=num_cores, num_subcores=num_subcores)
    nw = num_cores * num_subcores
    per_w = n_tokens // nw

    s1 = min(128, seq_len)
    s2 = seq_len - s1

    def body(ids_hbm, tt_hbm, word_hbm, pos_hbm, type_hbm, gamma_hbm,
             beta_hbm, out_hbm, idx_a, idx_b, tt_v, rows_v, pos_v,
             type_v, gamma_v, beta_v, sem_g, sem_o):
        wid = lax.axis_index("s") * num_cores + lax.axis_index("c")
        base_w = wid * per_w

        pltpu.sync_copy(pos_hbm.at[pl.ds(0, seq_len)], pos_v)
        pltpu.sync_copy(type_hbm, type_v)
        pltpu.sync_copy(gamma_hbm, gamma_v)
        pltpu.sync_copy(beta_hbm, beta_v)

        type0 = [type_v[0, pl.ds(j * LANES, LANES)] for j in range(NJ)]
        type1 = [type_v[1, pl.ds(j * LANES, LANES)] for j in range(NJ)]
        dtype_ = [type1[j] - type0[j] for j in range(NJ)]
        gam = [gamma_v[pl.ds(j * LANES, LANES)] for j in range(NJ)]
        bet = [beta_v[pl.ds(j * LANES, LANES)] for j in range(NJ)]

        @plsc.parallel_loop(0, seq_len, 1, unroll=4)
        def fold_type0(i):
            for j in range(NJ):
                pos_v[i, pl.ds(j * LANES, LANES)] = (
                    pos_v[i, pl.ds(j * LANES, LANES)] + type0[j])

        s2p = 80

        def issue_gather(ci):
            base = base_w + ci * seq_len
            slot = lax.rem(ci, 2)
            buf = lax.rem(ci, 3)
            pltpu.sync_copy(ids_hbm.at[pl.ds(base, s1)],
                            idx_a.at[pl.ds(slot * s1, s1)])
            pltpu.sync_copy(ids_hbm.at[pl.ds(base + s1, s2)],
                            idx_b.at[pl.ds(slot * s2p, s2)])
            pltpu.sync_copy(tt_hbm.at[pl.ds(base, seq_len)],
                            tt_v.at[pl.ds(slot * (seq_len + LANES), seq_len)])
            pltpu.async_copy(word_hbm.at[idx_a.at[pl.ds(slot * s1, s1)]],
                             rows_v.at[buf, pl.ds(0, s1)], sem_g)
            pltpu.async_copy(word_hbm.at[idx_b.at[pl.ds(slot * s2p, s2)]],
                             rows_v.at[buf, pl.ds(s1, s2)], sem_g)

        def wait_gather(ci):
            slot = lax.rem(ci, 2)
            buf = lax.rem(ci, 3)
            pltpu.make_async_copy(
                word_hbm.at[idx_a.at[pl.ds(slot * s1, s1)]],
                rows_v.at[buf, pl.ds(0, s1)], sem_g).wait()
            pltpu.make_async_copy(
                word_hbm.at[idx_b.at[pl.ds(slot * s2p, s2)]],
                rows_v.at[buf, pl.ds(s1, s2)], sem_g).wait()

        def wait_out(ci):
            buf = lax.rem(ci, 3)
            base = base_w + ci * seq_len
            pltpu.make_async_copy(rows_v.at[buf],
                                  out_hbm.at[pl.ds(base, seq_len)],
                                  sem_o).wait()

        issue_gather(0)

        def chunk_body(ci, carry):
            buf = lax.rem(ci, 3)
            slot = lax.rem(ci, 2)
            base = base_w + ci * seq_len

            @pl.when(ci >= 2)
            def _():
                wait_out(ci - 2)

            @pl.when(ci < n_chunks - 1)
            def _():
                issue_gather(ci + 1)

            wait_gather(ci)

            @plsc.parallel_loop(0, seq_len, 1, unroll=3,
                                carry=(dtype_, gam, bet))
            def tok_body(i, c):
                dt, gm, bt = c
                tf = tt_v[pl.ds(slot * (seq_len + LANES) + i,
                                LANES)][0].astype(jnp.float32)
                e = []
                for j in range(NJ):
                    ej = (rows_v[buf, i, pl.ds(j * LANES, LANES)]
                          + pos_v[i, pl.ds(j * LANES, LANES)]
                          + tf * dt[j])
                    e.append(ej)
                s = _tree_sum(e)
                q = _tree_sum([ej * ej for ej in e])
                mean = jnp.sum(s) * jnp.float32(1.0 / HIDDEN)
                meansq = jnp.sum(q) * jnp.float32(1.0 / HIDDEN)
                var = meansq - mean * mean
                rstd = _rsqrt(var + jnp.float32(EPS))
                for j in range(NJ):
                    rows_v[buf, i, pl.ds(j * LANES, LANES)] = (
                        (e[j] - mean) * (rstd * gm[j]) + bt[j])
                return c

            pltpu.async_copy(rows_v.at[buf], out_hbm.at[pl.ds(base, seq_len)],
                             sem_o)
            return carry

        lax.fori_loop(0, n_chunks, chunk_body, 0)
        wait_out(n_chunks - 2)
        wait_out(n_chunks - 1)

    return pl.kernel(
        body,
        out_type=jax.ShapeDtypeStruct((n_tokens, HIDDEN), jnp.float32),
        mesh=mesh,
        compiler_params=pltpu.CompilerParams(needs_layout_passes=False),
        scratch_types=[
            pltpu.VMEM((2 * s1,), jnp.int32),
            pltpu.VMEM((2 * 80,), jnp.int32),
            pltpu.VMEM((2 * (seq_len + LANES),), jnp.int32),
            pltpu.VMEM((3, seq_len, HIDDEN), jnp.float32),
            pltpu.VMEM((seq_len, HIDDEN), jnp.float32),
            pltpu.VMEM((2, HIDDEN), jnp.float32),
            pltpu.VMEM((HIDDEN,), jnp.float32),
            pltpu.VMEM((HIDDEN,), jnp.float32),
            pltpu.SemaphoreType.DMA,
            pltpu.SemaphoreType.DMA,
        ],
    )


@jax.jit
def kernel(input_ids, token_type_ids, word_emb, pos_emb, type_emb, gamma,
           beta):
    b, l = input_ids.shape
    n = b * l
    nw = 32
    n_chunks = (n // nw) // l
    ids = input_ids.reshape(n).astype(jnp.int32)
    tt = token_type_ids.reshape(n).astype(jnp.int32)
    k = _make_sc_kernel(n, l, n_chunks)
    out = k(ids, tt, word_emb, pos_emb, type_emb, gamma, beta)
    return out.reshape(b, l, HIDDEN)


# speedup vs baseline: 1.6331x; 1.0621x over previous
import functools

import jax
import jax.numpy as jnp
from jax import lax
from jax.experimental import pallas as pl
from jax.experimental.pallas import tpu as pltpu
from jax.experimental.pallas import tpu_sc as plsc

HIDDEN = 128
LANES = 16
NJ = HIDDEN // LANES
EPS = 1e-12


def _rsqrt(x):
    i = lax.bitcast_convert_type(x, jnp.int32)
    i = jnp.int32(0x5F3759DF) - (i >> 1)
    y = lax.bitcast_convert_type(i, jnp.float32)
    for _ in range(3):
        y = y * (jnp.float32(1.5) - jnp.float32(0.5) * x * y * y)
    return y


def _tree_sum(vs):
    vs = list(vs)
    while len(vs) > 1:
        nxt = [vs[k] + vs[k + 1] for k in range(0, len(vs) - 1, 2)]
        if len(vs) % 2:
            nxt.append(vs[-1])
        vs = nxt
    return vs[0]


def _make_sc_kernel(n_tokens, seq_len, n_chunks):
    num_cores, num_subcores = 2, 16
    mesh = plsc.VectorSubcoreMesh(
        core_axis_name="c", subcore_axis_name="s",
        num_cores=num_cores, num_subcores=num_subcores)
    nw = num_cores * num_subcores
    per_w = n_tokens // nw

    s1 = min(128, seq_len)
    s2 = seq_len - s1

    def body(ids_hbm, tt_hbm, word_hbm, pos_hbm, type_hbm, gamma_hbm,
             beta_hbm, out_hbm, idx_a, idx_b, tt_v, rows_v, pos_v,
             type_v, gamma_v, beta_v, sem_g, sem_o):
        wid = lax.axis_index("s") * num_cores + lax.axis_index("c")
        base_w = wid * per_w

        pltpu.sync_copy(pos_hbm.at[pl.ds(0, seq_len)], pos_v)
        pltpu.sync_copy(type_hbm, type_v)
        pltpu.sync_copy(gamma_hbm, gamma_v)
        pltpu.sync_copy(beta_hbm, beta_v)

        type0 = [type_v[0, pl.ds(j * LANES, LANES)] for j in range(NJ)]
        type1 = [type_v[1, pl.ds(j * LANES, LANES)] for j in range(NJ)]
        dtype_ = [type1[j] - type0[j] for j in range(NJ)]
        gam = [gamma_v[pl.ds(j * LANES, LANES)] for j in range(NJ)]
        bet = [beta_v[pl.ds(j * LANES, LANES)] for j in range(NJ)]

        @plsc.parallel_loop(0, seq_len, 1, unroll=4)
        def fold_type0(i):
            for j in range(NJ):
                pos_v[i, pl.ds(j * LANES, LANES)] = (
                    pos_v[i, pl.ds(j * LANES, LANES)] + type0[j])

        s2p = 80

        def issue_gather(ci):
            base = base_w + ci * seq_len
            slot = lax.rem(ci, 2)
            buf = lax.rem(ci, 3)
            pltpu.sync_copy(ids_hbm.at[pl.ds(base, s1)],
                            idx_a.at[pl.ds(slot * s1, s1)])
            pltpu.sync_copy(ids_hbm.at[pl.ds(base + s1, s2)],
                            idx_b.at[pl.ds(slot * s2p, s2)])
            pltpu.sync_copy(tt_hbm.at[pl.ds(base, seq_len)],
                            tt_v.at[pl.ds(slot * (seq_len + LANES), seq_len)])
            pltpu.async_copy(word_hbm.at[idx_a.at[pl.ds(slot * s1, s1)]],
                             rows_v.at[buf, pl.ds(0, s1)], sem_g)
            pltpu.async_copy(word_hbm.at[idx_b.at[pl.ds(slot * s2p, s2)]],
                             rows_v.at[buf, pl.ds(s1, s2)], sem_g)

        def wait_gather(ci):
            slot = lax.rem(ci, 2)
            buf = lax.rem(ci, 3)
            pltpu.make_async_copy(
                word_hbm.at[idx_a.at[pl.ds(slot * s1, s1)]],
                rows_v.at[buf, pl.ds(0, s1)], sem_g).wait()
            pltpu.make_async_copy(
                word_hbm.at[idx_b.at[pl.ds(slot * s2p, s2)]],
                rows_v.at[buf, pl.ds(s1, s2)], sem_g).wait()

        def wait_out(ci):
            buf = lax.rem(ci, 3)
            base = base_w + ci * seq_len
            pltpu.make_async_copy(rows_v.at[buf],
                                  out_hbm.at[pl.ds(base, seq_len)],
                                  sem_o).wait()

        issue_gather(0)

        def chunk_body(ci, carry):
            buf = lax.rem(ci, 3)
            slot = lax.rem(ci, 2)
            base = base_w + ci * seq_len

            @pl.when(ci >= 2)
            def _():
                wait_out(ci - 2)

            @pl.when(ci < n_chunks - 1)
            def _():
                issue_gather(ci + 1)

            wait_gather(ci)

            @plsc.parallel_loop(0, seq_len, 1, unroll=2,
                                carry=(dtype_, gam, bet))
            def tok_body(i, c):
                dt, gm, bt = c
                tf = tt_v[pl.ds(slot * (seq_len + LANES) + i,
                                LANES)][0].astype(jnp.float32)
                e = []
                for j in range(NJ):
                    ej = (rows_v[buf, i, pl.ds(j * LANES, LANES)]
                          + pos_v[i, pl.ds(j * LANES, LANES)]
                          + tf * dt[j])
                    e.append(ej)
                s = _tree_sum(e)
                q = _tree_sum([ej * ej for ej in e])
                mean = jnp.sum(s) * jnp.float32(1.0 / HIDDEN)
                meansq = jnp.sum(q) * jnp.float32(1.0 / HIDDEN)
                var = meansq - mean * mean
                rstd = _rsqrt(var + jnp.float32(EPS))
                for j in range(NJ):
                    rows_v[buf, i, pl.ds(j * LANES, LANES)] = (
                        (e[j] - mean) * (rstd * gm[j]) + bt[j])
                return c

            pltpu.async_copy(rows_v.at[buf], out_hbm.at[pl.ds(base, seq_len)],
                             sem_o)
            return carry

        lax.fori_loop(0, n_chunks, chunk_body, 0)
        wait_out(n_chunks - 2)
        wait_out(n_chunks - 1)

    return pl.kernel(
        body,
        out_type=jax.ShapeDtypeStruct((n_tokens, HIDDEN), jnp.float32),
        mesh=mesh,
        compiler_params=pltpu.CompilerParams(needs_layout_passes=False),
        scratch_types=[
            pltpu.VMEM((2 * s1,), jnp.int32),
            pltpu.VMEM((2 * 80,), jnp.int32),
            pltpu.VMEM((2 * (seq_len + LANES),), jnp.int32),
            pltpu.VMEM((3, seq_len, HIDDEN), jnp.float32),
            pltpu.VMEM((seq_len, HIDDEN), jnp.float32),
            pltpu.VMEM((2, HIDDEN), jnp.float32),
            pltpu.VMEM((HIDDEN,), jnp.float32),
            pltpu.VMEM((HIDDEN,), jnp.float32),
            pltpu.SemaphoreType.DMA,
            pltpu.SemaphoreType.DMA,
        ],
    )


@jax.jit
def kernel(input_ids, token_type_ids, word_emb, pos_emb, type_emb, gamma,
           beta):
    b, l = input_ids.shape
    n = b * l
    nw = 32
    n_chunks = (n // nw) // l
    ids = input_ids.reshape(n).astype(jnp.int32)
    tt = token_type_ids.reshape(n).astype(jnp.int32)
    k = _make_sc_kernel(n, l, n_chunks)
    out = k(ids, tt, word_emb, pos_emb, type_emb, gamma, beta)
    return out.reshape(b, l, HIDDEN)


# speedup vs baseline: 2.0758x; 1.2711x over previous
import functools

import jax
import jax.numpy as jnp
from jax import lax
from jax.experimental import pallas as pl
from jax.experimental.pallas import tpu as pltpu
from jax.experimental.pallas import tpu_sc as plsc

HIDDEN = 128
LANES = 16
NJ = HIDDEN // LANES
EPS = 1e-12


def _rsqrt(x):
    i = lax.bitcast_convert_type(x, jnp.int32)
    i = jnp.int32(0x5F3759DF) - (i >> 1)
    y = lax.bitcast_convert_type(i, jnp.float32)
    for _ in range(3):
        y = y * (jnp.float32(1.5) - jnp.float32(0.5) * x * y * y)
    return y


def _tree_sum(vs):
    vs = list(vs)
    while len(vs) > 1:
        nxt = [vs[k] + vs[k + 1] for k in range(0, len(vs) - 1, 2)]
        if len(vs) % 2:
            nxt.append(vs[-1])
        vs = nxt
    return vs[0]


def _make_sc_kernel(n_tokens, seq_len, n_chunks):
    num_cores, num_subcores = 2, 16
    mesh = plsc.VectorSubcoreMesh(
        core_axis_name="c", subcore_axis_name="s",
        num_cores=num_cores, num_subcores=num_subcores)
    nw = num_cores * num_subcores
    per_w = n_tokens // nw

    s1 = min(128, seq_len)
    s2 = seq_len - s1

    def body(ids_hbm, tt_hbm, word_hbm, pos_hbm, type_hbm, gamma_hbm,
             beta_hbm, out_hbm, idx_a, idx_b, tt_v, rows_v, pos_v,
             type_v, gamma_v, beta_v, sem_g, sem_o, sem_i):
        wid = lax.axis_index("s") * num_cores + lax.axis_index("c")
        base_w = wid * per_w

        pltpu.sync_copy(pos_hbm.at[pl.ds(0, seq_len)], pos_v)
        pltpu.sync_copy(type_hbm, type_v)
        pltpu.sync_copy(gamma_hbm, gamma_v)
        pltpu.sync_copy(beta_hbm, beta_v)

        type0 = [type_v[0, pl.ds(j * LANES, LANES)] for j in range(NJ)]
        type1 = [type_v[1, pl.ds(j * LANES, LANES)] for j in range(NJ)]
        dtype_ = [type1[j] - type0[j] for j in range(NJ)]
        gam = [gamma_v[pl.ds(j * LANES, LANES)] for j in range(NJ)]
        bet = [beta_v[pl.ds(j * LANES, LANES)] for j in range(NJ)]

        @plsc.parallel_loop(0, seq_len, 1, unroll=4)
        def fold_type0(i):
            for j in range(NJ):
                pos_v[i, pl.ds(j * LANES, LANES)] = (
                    pos_v[i, pl.ds(j * LANES, LANES)] + type0[j])

        s2p = 80
        ttp = seq_len + LANES

        def stage_idx(ci):
            base = base_w + ci * seq_len
            slot = lax.rem(ci, 3)
            pltpu.async_copy(ids_hbm.at[pl.ds(base, s1)],
                             idx_a.at[pl.ds(slot * s1, s1)], sem_i)
            pltpu.async_copy(ids_hbm.at[pl.ds(base + s1, s2)],
                             idx_b.at[pl.ds(slot * s2p, s2)], sem_i)
            pltpu.async_copy(tt_hbm.at[pl.ds(base, seq_len)],
                             tt_v.at[pl.ds(slot * ttp, seq_len)], sem_i)

        def wait_idx(ci):
            base = base_w + ci * seq_len
            slot = lax.rem(ci, 3)
            pltpu.make_async_copy(ids_hbm.at[pl.ds(base, s1)],
                                  idx_a.at[pl.ds(slot * s1, s1)],
                                  sem_i).wait()
            pltpu.make_async_copy(ids_hbm.at[pl.ds(base + s1, s2)],
                                  idx_b.at[pl.ds(slot * s2p, s2)],
                                  sem_i).wait()
            pltpu.make_async_copy(tt_hbm.at[pl.ds(base, seq_len)],
                                  tt_v.at[pl.ds(slot * ttp, seq_len)],
                                  sem_i).wait()

        def issue_gather(ci):
            slot = lax.rem(ci, 3)
            buf = lax.rem(ci, 3)
            pltpu.async_copy(word_hbm.at[idx_a.at[pl.ds(slot * s1, s1)]],
                             rows_v.at[buf, pl.ds(0, s1)], sem_g)
            pltpu.async_copy(word_hbm.at[idx_b.at[pl.ds(slot * s2p, s2)]],
                             rows_v.at[buf, pl.ds(s1, s2)], sem_g)

        def wait_gather(ci):
            slot = lax.rem(ci, 3)
            buf = lax.rem(ci, 3)
            pltpu.make_async_copy(
                word_hbm.at[idx_a.at[pl.ds(slot * s1, s1)]],
                rows_v.at[buf, pl.ds(0, s1)], sem_g).wait()
            pltpu.make_async_copy(
                word_hbm.at[idx_b.at[pl.ds(slot * s2p, s2)]],
                rows_v.at[buf, pl.ds(s1, s2)], sem_g).wait()

        def wait_out(ci):
            buf = lax.rem(ci, 3)
            base = base_w + ci * seq_len
            pltpu.make_async_copy(rows_v.at[buf],
                                  out_hbm.at[pl.ds(base, seq_len)],
                                  sem_o).wait()

        stage_idx(0)
        stage_idx(1)
        wait_idx(0)
        issue_gather(0)

        def chunk_body(ci, carry):
            buf = lax.rem(ci, 3)
            slot = lax.rem(ci, 3)
            base = base_w + ci * seq_len

            @pl.when(ci + 2 < n_chunks)
            def _():
                stage_idx(ci + 2)

            @pl.when(ci >= 2)
            def _():
                wait_out(ci - 2)

            @pl.when(ci < n_chunks - 1)
            def _():
                wait_idx(ci + 1)
                issue_gather(ci + 1)

            wait_gather(ci)

            @plsc.parallel_loop(0, seq_len, 1, unroll=2,
                                carry=(dtype_, gam, bet))
            def tok_body(i, c):
                dt, gm, bt = c
                tf = tt_v[pl.ds(slot * ttp + i,
                                LANES)][0].astype(jnp.float32)
                e = []
                for j in range(NJ):
                    ej = (rows_v[buf, i, pl.ds(j * LANES, LANES)]
                          + pos_v[i, pl.ds(j * LANES, LANES)]
                          + tf * dt[j])
                    e.append(ej)
                s = _tree_sum(e)
                q = _tree_sum([ej * ej for ej in e])
                mean = jnp.sum(s) * jnp.float32(1.0 / HIDDEN)
                meansq = jnp.sum(q) * jnp.float32(1.0 / HIDDEN)
                var = meansq - mean * mean
                rstd = _rsqrt(var + jnp.float32(EPS))
                for j in range(NJ):
                    rows_v[buf, i, pl.ds(j * LANES, LANES)] = (
                        (e[j] - mean) * (rstd * gm[j]) + bt[j])
                return c

            pltpu.async_copy(rows_v.at[buf], out_hbm.at[pl.ds(base, seq_len)],
                             sem_o)
            return carry

        lax.fori_loop(0, n_chunks, chunk_body, 0)
        wait_out(n_chunks - 2)
        wait_out(n_chunks - 1)

    return pl.kernel(
        body,
        out_type=jax.ShapeDtypeStruct((n_tokens, HIDDEN), jnp.float32),
        mesh=mesh,
        compiler_params=pltpu.CompilerParams(needs_layout_passes=False),
        scratch_types=[
            pltpu.VMEM((3 * s1,), jnp.int32),
            pltpu.VMEM((3 * 80,), jnp.int32),
            pltpu.VMEM((3 * (seq_len + LANES),), jnp.int32),
            pltpu.VMEM((3, seq_len, HIDDEN), jnp.float32),
            pltpu.VMEM((seq_len, HIDDEN), jnp.float32),
            pltpu.VMEM((2, HIDDEN), jnp.float32),
            pltpu.VMEM((HIDDEN,), jnp.float32),
            pltpu.VMEM((HIDDEN,), jnp.float32),
            pltpu.SemaphoreType.DMA,
            pltpu.SemaphoreType.DMA,
            pltpu.SemaphoreType.DMA,
        ],
    )


@jax.jit
def kernel(input_ids, token_type_ids, word_emb, pos_emb, type_emb, gamma,
           beta):
    b, l = input_ids.shape
    n = b * l
    nw = 32
    n_chunks = (n // nw) // l
    ids = input_ids.reshape(n).astype(jnp.int32)
    tt = token_type_ids.reshape(n).astype(jnp.int32)
    k = _make_sc_kernel(n, l, n_chunks)
    out = k(ids, tt, word_emb, pos_emb, type_emb, gamma, beta)
    return out.reshape(b, l, HIDDEN)


# speedup vs baseline: 2.2508x; 1.0843x over previous
import functools

import jax
import jax.numpy as jnp
from jax import lax
from jax.experimental import pallas as pl
from jax.experimental.pallas import tpu as pltpu
from jax.experimental.pallas import tpu_sc as plsc

HIDDEN = 128
LANES = 16
NJ = HIDDEN // LANES
EPS = 1e-12


def _rsqrt(x):
    i = lax.bitcast_convert_type(x, jnp.int32)
    i = jnp.int32(0x5F3759DF) - (i >> 1)
    y = lax.bitcast_convert_type(i, jnp.float32)
    for _ in range(3):
        y = y * (jnp.float32(1.5) - jnp.float32(0.5) * x * y * y)
    return y


def _tree_sum(vs):
    vs = list(vs)
    while len(vs) > 1:
        nxt = [vs[k] + vs[k + 1] for k in range(0, len(vs) - 1, 2)]
        if len(vs) % 2:
            nxt.append(vs[-1])
        vs = nxt
    return vs[0]


def _make_sc_kernel(n_tokens, seq_len, n_chunks):
    num_cores, num_subcores = 2, 16
    mesh = plsc.VectorSubcoreMesh(
        core_axis_name="c", subcore_axis_name="s",
        num_cores=num_cores, num_subcores=num_subcores)
    nw = num_cores * num_subcores
    per_w = n_tokens // nw

    s1 = min(128, seq_len)
    s2 = seq_len - s1

    def body(ids_hbm, tt_hbm, word_hbm, pos_hbm, type_hbm, gamma_hbm,
             beta_hbm, out_hbm, idx_a, idx_b, tt_v, rows_v, pos_v,
             type_v, gamma_v, beta_v, sem_g, sem_o, sem_i):
        wid = lax.axis_index("s") * num_cores + lax.axis_index("c")
        base_w = wid * per_w

        pltpu.sync_copy(pos_hbm.at[pl.ds(0, seq_len)], pos_v)
        pltpu.sync_copy(type_hbm, type_v)
        pltpu.sync_copy(gamma_hbm, gamma_v)
        pltpu.sync_copy(beta_hbm, beta_v)

        type0 = [type_v[0, pl.ds(j * LANES, LANES)] for j in range(NJ)]
        type1 = [type_v[1, pl.ds(j * LANES, LANES)] for j in range(NJ)]
        dtype_ = [type1[j] - type0[j] for j in range(NJ)]
        gam = [gamma_v[pl.ds(j * LANES, LANES)] for j in range(NJ)]
        bet = [beta_v[pl.ds(j * LANES, LANES)] for j in range(NJ)]

        @plsc.parallel_loop(0, seq_len, 1, unroll=4)
        def fold_type0(i):
            for j in range(NJ):
                pos_v[i, pl.ds(j * LANES, LANES)] = (
                    pos_v[i, pl.ds(j * LANES, LANES)] + type0[j])

        s2p = 80
        ttp = seq_len + LANES

        def stage_idx(ci):
            base = base_w + ci * seq_len
            slot = lax.rem(ci, 4)
            pltpu.async_copy(ids_hbm.at[pl.ds(base, s1)],
                             idx_a.at[pl.ds(slot * s1, s1)], sem_i)
            pltpu.async_copy(ids_hbm.at[pl.ds(base + s1, s2)],
                             idx_b.at[pl.ds(slot * s2p, s2)], sem_i)
            pltpu.async_copy(tt_hbm.at[pl.ds(base, seq_len)],
                             tt_v.at[pl.ds(slot * ttp, seq_len)], sem_i)

        def wait_idx(ci):
            base = base_w + ci * seq_len
            slot = lax.rem(ci, 4)
            pltpu.make_async_copy(ids_hbm.at[pl.ds(base, s1)],
                                  idx_a.at[pl.ds(slot * s1, s1)],
                                  sem_i).wait()
            pltpu.make_async_copy(ids_hbm.at[pl.ds(base + s1, s2)],
                                  idx_b.at[pl.ds(slot * s2p, s2)],
                                  sem_i).wait()
            pltpu.make_async_copy(tt_hbm.at[pl.ds(base, seq_len)],
                                  tt_v.at[pl.ds(slot * ttp, seq_len)],
                                  sem_i).wait()

        def issue_gather(ci):
            slot = lax.rem(ci, 4)
            buf = lax.rem(ci, 4)
            pltpu.async_copy(word_hbm.at[idx_a.at[pl.ds(slot * s1, s1)]],
                             rows_v.at[buf, pl.ds(0, s1)], sem_g)
            pltpu.async_copy(word_hbm.at[idx_b.at[pl.ds(slot * s2p, s2)]],
                             rows_v.at[buf, pl.ds(s1, s2)], sem_g)

        def wait_gather(ci):
            slot = lax.rem(ci, 4)
            buf = lax.rem(ci, 4)
            pltpu.make_async_copy(
                word_hbm.at[idx_a.at[pl.ds(slot * s1, s1)]],
                rows_v.at[buf, pl.ds(0, s1)], sem_g).wait()
            pltpu.make_async_copy(
                word_hbm.at[idx_b.at[pl.ds(slot * s2p, s2)]],
                rows_v.at[buf, pl.ds(s1, s2)], sem_g).wait()

        def wait_out(ci):
            buf = lax.rem(ci, 4)
            base = base_w + ci * seq_len
            pltpu.make_async_copy(rows_v.at[buf],
                                  out_hbm.at[pl.ds(base, seq_len)],
                                  sem_o).wait()

        stage_idx(0)
        stage_idx(1)
        stage_idx(2)
        wait_idx(0)
        issue_gather(0)
        wait_idx(1)
        issue_gather(1)

        def chunk_body(ci, carry):
            buf = lax.rem(ci, 4)
            slot = lax.rem(ci, 4)
            base = base_w + ci * seq_len

            @pl.when(ci + 3 < n_chunks)
            def _():
                stage_idx(ci + 3)

            @pl.when(ci >= 2)
            def _():
                wait_out(ci - 2)

            @pl.when(ci + 2 < n_chunks)
            def _():
                wait_idx(ci + 2)
                issue_gather(ci + 2)

            wait_gather(ci)

            @plsc.parallel_loop(0, seq_len, 1, unroll=2,
                                carry=(dtype_, gam, bet))
            def tok_body(i, c):
                dt, gm, bt = c
                tf = tt_v[pl.ds(slot * ttp + i,
                                LANES)][0].astype(jnp.float32)
                e = []
                for j in range(NJ):
                    ej = (rows_v[buf, i, pl.ds(j * LANES, LANES)]
                          + pos_v[i, pl.ds(j * LANES, LANES)]
                          + tf * dt[j])
                    e.append(ej)
                s = _tree_sum(e)
                q = _tree_sum([ej * ej for ej in e])
                mean = jnp.sum(s) * jnp.float32(1.0 / HIDDEN)
                meansq = jnp.sum(q) * jnp.float32(1.0 / HIDDEN)
                var = meansq - mean * mean
                rstd = _rsqrt(var + jnp.float32(EPS))
                for j in range(NJ):
                    rows_v[buf, i, pl.ds(j * LANES, LANES)] = (
                        (e[j] - mean) * (rstd * gm[j]) + bt[j])
                return c

            pltpu.async_copy(rows_v.at[buf], out_hbm.at[pl.ds(base, seq_len)],
                             sem_o)
            return carry

        lax.fori_loop(0, n_chunks, chunk_body, 0)
        wait_out(n_chunks - 2)
        wait_out(n_chunks - 1)

    return pl.kernel(
        body,
        out_type=jax.ShapeDtypeStruct((n_tokens, HIDDEN), jnp.float32),
        mesh=mesh,
        compiler_params=pltpu.CompilerParams(needs_layout_passes=False),
        scratch_types=[
            pltpu.VMEM((4 * s1,), jnp.int32),
            pltpu.VMEM((4 * 80,), jnp.int32),
            pltpu.VMEM((4 * (seq_len + LANES),), jnp.int32),
            pltpu.VMEM((4, seq_len, HIDDEN), jnp.float32),
            pltpu.VMEM((seq_len, HIDDEN), jnp.float32),
            pltpu.VMEM((2, HIDDEN), jnp.float32),
            pltpu.VMEM((HIDDEN,), jnp.float32),
            pltpu.VMEM((HIDDEN,), jnp.float32),
            pltpu.SemaphoreType.DMA,
            pltpu.SemaphoreType.DMA,
            pltpu.SemaphoreType.DMA,
        ],
    )


@jax.jit
def kernel(input_ids, token_type_ids, word_emb, pos_emb, type_emb, gamma,
           beta):
    b, l = input_ids.shape
    n = b * l
    nw = 32
    n_chunks = (n // nw) // l
    ids = input_ids.reshape(n).astype(jnp.int32)
    tt = token_type_ids.reshape(n).astype(jnp.int32)
    k = _make_sc_kernel(n, l, n_chunks)
    out = k(ids, tt, word_emb, pos_emb, type_emb, gamma, beta)
    return out.reshape(b, l, HIDDEN)


# speedup vs baseline: 2.2572x; 1.0028x over previous
import functools

import jax
import jax.numpy as jnp
from jax import lax
from jax.experimental import pallas as pl
from jax.experimental.pallas import tpu as pltpu
from jax.experimental.pallas import tpu_sc as plsc

HIDDEN = 128
LANES = 16
NJ = HIDDEN // LANES
EPS = 1e-12


def _rsqrt(x):
    i = lax.bitcast_convert_type(x, jnp.int32)
    i = jnp.int32(0x5F3759DF) - (i >> 1)
    y = lax.bitcast_convert_type(i, jnp.float32)
    for _ in range(3):
        y = y * (jnp.float32(1.5) - jnp.float32(0.5) * x * y * y)
    return y


def _tree_sum(vs):
    vs = list(vs)
    while len(vs) > 1:
        nxt = [vs[k] + vs[k + 1] for k in range(0, len(vs) - 1, 2)]
        if len(vs) % 2:
            nxt.append(vs[-1])
        vs = nxt
    return vs[0]


def _make_sc_kernel(n_tokens, seq_len, n_chunks):
    num_cores, num_subcores = 2, 16
    mesh = plsc.VectorSubcoreMesh(
        core_axis_name="c", subcore_axis_name="s",
        num_cores=num_cores, num_subcores=num_subcores)
    nw = num_cores * num_subcores
    per_w = n_tokens // nw

    s1 = min(128, seq_len)
    s2 = seq_len - s1

    def body(ids_hbm, tt_hbm, word_hbm, pos_hbm, type_hbm, gamma_hbm,
             beta_hbm, out_hbm, idx_a, idx_b, tt_v, rows_v, pos_v,
             type_v, gamma_v, beta_v, sem_g, sem_o, sem_i):
        wid = lax.axis_index("s") * num_cores + lax.axis_index("c")
        base_w = wid * per_w

        pltpu.sync_copy(pos_hbm.at[pl.ds(0, seq_len)], pos_v)
        pltpu.sync_copy(type_hbm, type_v)
        pltpu.sync_copy(gamma_hbm, gamma_v)
        pltpu.sync_copy(beta_hbm, beta_v)

        type0 = [type_v[0, pl.ds(j * LANES, LANES)] for j in range(NJ)]
        type1 = [type_v[1, pl.ds(j * LANES, LANES)] for j in range(NJ)]
        dtype_ = [type1[j] - type0[j] for j in range(NJ)]
        gam = [gamma_v[pl.ds(j * LANES, LANES)] for j in range(NJ)]
        bet = [beta_v[pl.ds(j * LANES, LANES)] for j in range(NJ)]

        s2p = 80
        ttp = seq_len + LANES

        def stage_idx(ci):
            base = base_w + ci * seq_len
            slot = lax.rem(ci, 4)
            pltpu.async_copy(ids_hbm.at[pl.ds(base, s1)],
                             idx_a.at[pl.ds(slot * s1, s1)], sem_i)
            pltpu.async_copy(ids_hbm.at[pl.ds(base + s1, s2)],
                             idx_b.at[pl.ds(slot * s2p, s2)], sem_i)
            pltpu.async_copy(tt_hbm.at[pl.ds(base, seq_len)],
                             tt_v.at[pl.ds(slot * ttp, seq_len)], sem_i)

        def wait_idx(ci):
            base = base_w + ci * seq_len
            slot = lax.rem(ci, 4)
            pltpu.make_async_copy(ids_hbm.at[pl.ds(base, s1)],
                                  idx_a.at[pl.ds(slot * s1, s1)],
                                  sem_i).wait()
            pltpu.make_async_copy(ids_hbm.at[pl.ds(base + s1, s2)],
                                  idx_b.at[pl.ds(slot * s2p, s2)],
                                  sem_i).wait()
            pltpu.make_async_copy(tt_hbm.at[pl.ds(base, seq_len)],
                                  tt_v.at[pl.ds(slot * ttp, seq_len)],
                                  sem_i).wait()

        def issue_gather(ci):
            slot = lax.rem(ci, 4)
            buf = lax.rem(ci, 4)
            pltpu.async_copy(word_hbm.at[idx_a.at[pl.ds(slot * s1, s1)]],
                             rows_v.at[buf, pl.ds(0, s1)], sem_g)
            pltpu.async_copy(word_hbm.at[idx_b.at[pl.ds(slot * s2p, s2)]],
                             rows_v.at[buf, pl.ds(s1, s2)], sem_g)

        def wait_gather(ci):
            slot = lax.rem(ci, 4)
            buf = lax.rem(ci, 4)
            pltpu.make_async_copy(
                word_hbm.at[idx_a.at[pl.ds(slot * s1, s1)]],
                rows_v.at[buf, pl.ds(0, s1)], sem_g).wait()
            pltpu.make_async_copy(
                word_hbm.at[idx_b.at[pl.ds(slot * s2p, s2)]],
                rows_v.at[buf, pl.ds(s1, s2)], sem_g).wait()

        def wait_out(ci):
            buf = lax.rem(ci, 4)
            base = base_w + ci * seq_len
            pltpu.make_async_copy(rows_v.at[buf],
                                  out_hbm.at[pl.ds(base, seq_len)],
                                  sem_o).wait()

        stage_idx(0)
        stage_idx(1)
        stage_idx(2)
        wait_idx(0)
        issue_gather(0)
        wait_idx(1)
        issue_gather(1)

        @plsc.parallel_loop(0, seq_len, 1, unroll=4)
        def fold_type0(i):
            for j in range(NJ):
                pos_v[i, pl.ds(j * LANES, LANES)] = (
                    pos_v[i, pl.ds(j * LANES, LANES)] + type0[j])

        def chunk_body(ci, carry):
            buf = lax.rem(ci, 4)
            slot = lax.rem(ci, 4)
            base = base_w + ci * seq_len

            @pl.when(ci + 3 < n_chunks)
            def _():
                stage_idx(ci + 3)

            @pl.when(ci >= 2)
            def _():
                wait_out(ci - 2)

            @pl.when(ci + 2 < n_chunks)
            def _():
                wait_idx(ci + 2)
                issue_gather(ci + 2)

            wait_gather(ci)

            @plsc.parallel_loop(0, seq_len, 1, unroll=2,
                                carry=(dtype_, gam, bet))
            def tok_body(i, c):
                dt, gm, bt = c
                tf = tt_v[pl.ds(slot * ttp + i,
                                LANES)][0].astype(jnp.float32)
                e = []
                for j in range(NJ):
                    ej = (rows_v[buf, i, pl.ds(j * LANES, LANES)]
                          + pos_v[i, pl.ds(j * LANES, LANES)]
                          + tf * dt[j])
                    e.append(ej)
                s = _tree_sum(e)
                q = _tree_sum([ej * ej for ej in e])
                mean = jnp.sum(s) * jnp.float32(1.0 / HIDDEN)
                meansq = jnp.sum(q) * jnp.float32(1.0 / HIDDEN)
                var = meansq - mean * mean
                rstd = _rsqrt(var + jnp.float32(EPS))
                for j in range(NJ):
                    rows_v[buf, i, pl.ds(j * LANES, LANES)] = (
                        (e[j] - mean) * (rstd * gm[j]) + bt[j])
                return c

            pltpu.async_copy(rows_v.at[buf], out_hbm.at[pl.ds(base, seq_len)],
                             sem_o)
            return carry

        lax.fori_loop(0, n_chunks, chunk_body, 0)
        wait_out(n_chunks - 2)
        wait_out(n_chunks - 1)

    return pl.kernel(
        body,
        out_type=jax.ShapeDtypeStruct((n_tokens, HIDDEN), jnp.float32),
        mesh=mesh,
        compiler_params=pltpu.CompilerParams(needs_layout_passes=False),
        scratch_types=[
            pltpu.VMEM((4 * s1,), jnp.int32),
            pltpu.VMEM((4 * 80,), jnp.int32),
            pltpu.VMEM((4 * (seq_len + LANES),), jnp.int32),
            pltpu.VMEM((4, seq_len, HIDDEN), jnp.float32),
            pltpu.VMEM((seq_len, HIDDEN), jnp.float32),
            pltpu.VMEM((2, HIDDEN), jnp.float32),
            pltpu.VMEM((HIDDEN,), jnp.float32),
            pltpu.VMEM((HIDDEN,), jnp.float32),
            pltpu.SemaphoreType.DMA,
            pltpu.SemaphoreType.DMA,
            pltpu.SemaphoreType.DMA,
        ],
    )


@jax.jit
def kernel(input_ids, token_type_ids, word_emb, pos_emb, type_emb, gamma,
           beta):
    b, l = input_ids.shape
    n = b * l
    nw = 32
    n_chunks = (n // nw) // l
    ids = input_ids.reshape(n).astype(jnp.int32)
    tt = token_type_ids.reshape(n).astype(jnp.int32)
    k = _make_sc_kernel(n, l, n_chunks)
    out = k(ids, tt, word_emb, pos_emb, type_emb, gamma, beta)
    return out.reshape(b, l, HIDDEN)


# speedup vs baseline: 2.7346x; 1.2115x over previous
import functools

import jax
import jax.numpy as jnp
from jax import lax
from jax.experimental import pallas as pl
from jax.experimental.pallas import tpu as pltpu
from jax.experimental.pallas import tpu_sc as plsc

HIDDEN = 128
LANES = 16
NJ = HIDDEN // LANES
EPS = 1e-12


def _rsqrt(x):
    i = lax.bitcast_convert_type(x, jnp.int32)
    i = jnp.int32(0x5F3759DF) - (i >> 1)
    y = lax.bitcast_convert_type(i, jnp.float32)
    for _ in range(3):
        y = y * (jnp.float32(1.5) - jnp.float32(0.5) * x * y * y)
    return y


def _tree_sum(vs):
    vs = list(vs)
    while len(vs) > 1:
        nxt = [vs[k] + vs[k + 1] for k in range(0, len(vs) - 1, 2)]
        if len(vs) % 2:
            nxt.append(vs[-1])
        vs = nxt
    return vs[0]


def _make_sc_kernel(n_tokens, seq_len, n_chunks):
    num_cores, num_subcores = 2, 16
    mesh = plsc.VectorSubcoreMesh(
        core_axis_name="c", subcore_axis_name="s",
        num_cores=num_cores, num_subcores=num_subcores)
    nw = num_cores * num_subcores
    per_w = n_tokens // nw

    s1 = min(128, seq_len)
    s2 = seq_len - s1

    def body(ids_hbm, tt_hbm, word_hbm, pos_hbm, type_hbm, gamma_hbm,
             beta_hbm, out_hbm, idx_a, idx_b, tt_v, rows_v, pos_v,
             type_v, gamma_v, beta_v, sem_g, sem_o, sem_i):
        wid = lax.axis_index("s") * num_cores + lax.axis_index("c")
        base_w = wid * per_w

        s2p = 80
        ttp = seq_len + LANES

        def stage_idx(ci):
            base = base_w + ci * seq_len
            slot = lax.rem(ci, 3)
            pltpu.async_copy(ids_hbm.at[pl.ds(base, s1)],
                             idx_a.at[pl.ds(slot * s1, s1)], sem_i)
            pltpu.async_copy(ids_hbm.at[pl.ds(base + s1, s2)],
                             idx_b.at[pl.ds(slot * s2p, s2)], sem_i)
            pltpu.async_copy(tt_hbm.at[pl.ds(base, seq_len)],
                             tt_v.at[pl.ds(slot * ttp, seq_len)], sem_i)

        def wait_idx(ci):
            base = base_w + ci * seq_len
            slot = lax.rem(ci, 3)
            pltpu.make_async_copy(ids_hbm.at[pl.ds(base, s1)],
                                  idx_a.at[pl.ds(slot * s1, s1)],
                                  sem_i).wait()
            pltpu.make_async_copy(ids_hbm.at[pl.ds(base + s1, s2)],
                                  idx_b.at[pl.ds(slot * s2p, s2)],
                                  sem_i).wait()
            pltpu.make_async_copy(tt_hbm.at[pl.ds(base, seq_len)],
                                  tt_v.at[pl.ds(slot * ttp, seq_len)],
                                  sem_i).wait()

        def issue_gather(ci):
            slot = lax.rem(ci, 3)
            buf = lax.rem(ci, 3)
            pltpu.async_copy(word_hbm.at[idx_a.at[pl.ds(slot * s1, s1)]],
                             rows_v.at[buf, pl.ds(0, s1)], sem_g)
            pltpu.async_copy(word_hbm.at[idx_b.at[pl.ds(slot * s2p, s2)]],
                             rows_v.at[buf, pl.ds(s1, s2)], sem_g)

        def wait_gather(ci):
            slot = lax.rem(ci, 3)
            buf = lax.rem(ci, 3)
            pltpu.make_async_copy(
                word_hbm.at[idx_a.at[pl.ds(slot * s1, s1)]],
                rows_v.at[buf, pl.ds(0, s1)], sem_g).wait()
            pltpu.make_async_copy(
                word_hbm.at[idx_b.at[pl.ds(slot * s2p, s2)]],
                rows_v.at[buf, pl.ds(s1, s2)], sem_g).wait()

        def wait_out(ci):
            buf = lax.rem(ci, 3)
            base = base_w + ci * seq_len
            pltpu.make_async_copy(rows_v.at[buf],
                                  out_hbm.at[pl.ds(base, seq_len)],
                                  sem_o).wait()

        pltpu.sync_copy(pos_hbm.at[pl.ds(0, seq_len)],
                        pos_v.at[pl.ds(0, seq_len)])
        pltpu.sync_copy(pos_hbm.at[pl.ds(0, seq_len)],
                        pos_v.at[pl.ds(seq_len, seq_len)])
        pltpu.sync_copy(type_hbm, type_v)
        pltpu.sync_copy(gamma_hbm, gamma_v)
        pltpu.sync_copy(beta_hbm, beta_v)

        type0 = [type_v[0, pl.ds(j * LANES, LANES)] for j in range(NJ)]
        type1 = [type_v[1, pl.ds(j * LANES, LANES)] for j in range(NJ)]
        gam = [gamma_v[pl.ds(j * LANES, LANES)] for j in range(NJ)]
        bet = [beta_v[pl.ds(j * LANES, LANES)] for j in range(NJ)]

        stage_idx(0)
        stage_idx(1)
        wait_idx(0)
        issue_gather(0)

        @plsc.parallel_loop(0, seq_len, 1, unroll=4)
        def fold_type(i):
            for j in range(NJ):
                pos_v[i, pl.ds(j * LANES, LANES)] = (
                    pos_v[i, pl.ds(j * LANES, LANES)] + type0[j])
            for j in range(NJ):
                pos_v[seq_len + i, pl.ds(j * LANES, LANES)] = (
                    pos_v[seq_len + i, pl.ds(j * LANES, LANES)] + type1[j])

        def chunk_body(ci, carry):
            buf = lax.rem(ci, 3)
            slot = lax.rem(ci, 3)
            base = base_w + ci * seq_len

            @pl.when(ci + 2 < n_chunks)
            def _():
                stage_idx(ci + 2)

            @pl.when(ci >= 2)
            def _():
                wait_out(ci - 2)

            @pl.when(ci + 1 < n_chunks)
            def _():
                wait_idx(ci + 1)
                issue_gather(ci + 1)

            wait_gather(ci)

            @plsc.parallel_loop(0, seq_len, 1, unroll=2,
                                carry=(gam, bet))
            def tok_body(i, c):
                gm, bt = c
                tk = tt_v[pl.ds(slot * ttp + i, LANES)][0]
                pi = i + tk * seq_len
                e = []
                for j in range(NJ):
                    ej = (rows_v[buf, i, pl.ds(j * LANES, LANES)]
                          + pos_v[pi, pl.ds(j * LANES, LANES)])
                    e.append(ej)
                s = _tree_sum(e)
                q = _tree_sum([ej * ej for ej in e])
                mean = jnp.sum(s) * jnp.float32(1.0 / HIDDEN)
                meansq = jnp.sum(q) * jnp.float32(1.0 / HIDDEN)
                var = meansq - mean * mean
                rstd = _rsqrt(var + jnp.float32(EPS))
                for j in range(NJ):
                    rows_v[buf, i, pl.ds(j * LANES, LANES)] = (
                        (e[j] - mean) * (rstd * gm[j]) + bt[j])
                return c

            pltpu.async_copy(rows_v.at[buf], out_hbm.at[pl.ds(base, seq_len)],
                             sem_o)
            return carry

        lax.fori_loop(0, n_chunks, chunk_body, 0)
        wait_out(n_chunks - 2)
        wait_out(n_chunks - 1)

    return pl.kernel(
        body,
        out_type=jax.ShapeDtypeStruct((n_tokens, HIDDEN), jnp.float32),
        mesh=mesh,
        compiler_params=pltpu.CompilerParams(needs_layout_passes=False),
        scratch_types=[
            pltpu.VMEM((3 * s1,), jnp.int32),
            pltpu.VMEM((3 * 80,), jnp.int32),
            pltpu.VMEM((3 * (seq_len + LANES),), jnp.int32),
            pltpu.VMEM((3, seq_len, HIDDEN), jnp.float32),
            pltpu.VMEM((2 * seq_len, HIDDEN), jnp.float32),
            pltpu.VMEM((2, HIDDEN), jnp.float32),
            pltpu.VMEM((HIDDEN,), jnp.float32),
            pltpu.VMEM((HIDDEN,), jnp.float32),
            pltpu.SemaphoreType.DMA,
            pltpu.SemaphoreType.DMA,
            pltpu.SemaphoreType.DMA,
        ],
    )


@jax.jit
def kernel(input_ids, token_type_ids, word_emb, pos_emb, type_emb, gamma,
           beta):
    b, l = input_ids.shape
    n = b * l
    nw = 32
    n_chunks = (n // nw) // l
    ids = input_ids.reshape(n).astype(jnp.int32)
    tt = token_type_ids.reshape(n).astype(jnp.int32)
    k = _make_sc_kernel(n, l, n_chunks)
    out = k(ids, tt, word_emb, pos_emb, type_emb, gamma, beta)
    return out.reshape(b, l, HIDDEN)
